# speedup vs baseline: 17.5292x; 17.5292x over previous
import jax
import jax.numpy as jnp
from jax.experimental import pallas as pl
from jax.experimental.pallas import tpu as pltpu

_SCORE_THRESH = 0.05
_NMS_THRESH = 0.5
_MAX_DET = 100
_LANES = 128


def _nms_body(x1, y1, x2, y2, sc, out, s_ref, area_ref):
    R, C = sc.shape
    s_ref[:] = jnp.where(sc[:] > _SCORE_THRESH, sc[:], -1.0)
    area_ref[:] = (x2[:] - x1[:]) * (y2[:] - y1[:])
    rowi = jax.lax.broadcasted_iota(jnp.int32, (R, C), 0)
    coli = jax.lax.broadcasted_iota(jnp.int32, (R, C), 1)
    lin = rowi * C + coli
    lane = jax.lax.broadcasted_iota(jnp.int32, (1, C), 1)
    neg = jnp.float32(-3.4e38)

    def body(i, carry):
        s = s_ref[:]
        m = jnp.max(s)
        bi = jnp.min(jnp.where(s >= m, lin, jnp.int32(2**30)))
        oh = lin == bi
        bx1 = jnp.max(jnp.where(oh, x1[:], neg))
        by1 = jnp.max(jnp.where(oh, y1[:], neg))
        bx2 = jnp.max(jnp.where(oh, x2[:], neg))
        by2 = jnp.max(jnp.where(oh, y2[:], neg))
        ba = jnp.max(jnp.where(oh, area_ref[:], neg))
        w = jnp.maximum(jnp.minimum(bx2, x2[:]) - jnp.maximum(bx1, x1[:]), 0.0)
        h = jnp.maximum(jnp.minimum(by2, y2[:]) - jnp.maximum(by1, y1[:]), 0.0)
        inter = w * h
        iou = inter / (ba + area_ref[:] - inter + 1e-9)
        s_ref[:] = jnp.where((iou > _NMS_THRESH) | oh, -1.0, s)
        valid = m > 0.0
        row = jnp.where(lane == 0, bx1,
              jnp.where(lane == 1, by1,
              jnp.where(lane == 2, bx2,
              jnp.where(lane == 3, by2,
              jnp.where(lane == 4, m, 0.0)))))
        row = jnp.where(valid, row, 0.0)
        out[pl.ds(i, 1), :] = row
        return carry

    jax.lax.fori_loop(0, _MAX_DET, body, 0)


def kernel(boxes, scores):
    n = boxes.shape[0]
    c = _LANES
    r = (n + c - 1) // c
    r = ((r + 7) // 8) * 8
    pad = r * c - n
    b = jnp.pad(boxes, ((0, pad), (0, 0)))
    s = jnp.pad(scores, (0, pad))
    x1 = b[:, 0].reshape(r, c)
    y1 = b[:, 1].reshape(r, c)
    x2 = b[:, 2].reshape(r, c)
    y2 = b[:, 3].reshape(r, c)
    s2 = s.reshape(r, c)
    out = pl.pallas_call(
        _nms_body,
        out_shape=jax.ShapeDtypeStruct((_MAX_DET, c), jnp.float32),
        scratch_shapes=[
            pltpu.VMEM((r, c), jnp.float32),
            pltpu.VMEM((r, c), jnp.float32),
        ],
    )(x1, y1, x2, y2, s2)
    return out[:, :5]
